# baseline (speedup 1.0000x reference)
import functools

import jax
import jax.numpy as jnp
from jax import lax
from jax.experimental import pallas as pl
from jax.experimental.pallas import tpu as pltpu
from jax.experimental.pallas import tpu_sc as plsc

N_TOK = 32768
D_MODEL = 1024
N_EXPERTS = 64
NUM_SELECTS = 8
BLOCK_T = 4096

_SC_CORES = 2
_SC_SUBCORES = 16
_SC_WORKERS = _SC_CORES * _SC_SUBCORES
_TOK_PER_W = N_TOK // _SC_WORKERS
_PAIRS_PER_W = _TOK_PER_W * NUM_SELECTS
_LANES = 16
_HIST = _LANES * N_EXPERTS


def _gate_body(x_ref, w1_ref, w2_ref, idx_ref, scr_ref):
    h = jnp.tanh(jnp.dot(x_ref[...], w1_ref[...],
                         preferred_element_type=jnp.float32))
    lt = jax.lax.dot_general(w2_ref[...], h, (((1,), (1,)), ((), ())),
                             preferred_element_type=jnp.float32)
    ne, bt = lt.shape
    iota_e = jax.lax.broadcasted_iota(jnp.int32, (ne, bt), 0)

    l = lt
    idx_rows, val_rows = [], []
    for _ in range(NUM_SELECTS):
        m = jnp.max(l, axis=0, keepdims=True)
        cand = jnp.where(l == m, iota_e, ne)
        idx = jnp.min(cand, axis=0, keepdims=True)
        onehot = iota_e == idx
        idx_rows.append(idx)
        val_rows.append(m)
        l = jnp.where(onehot, -jnp.inf, l)

    vals = jnp.concatenate(val_rows, axis=0)
    idxs = jnp.concatenate(idx_rows, axis=0)
    e = jnp.exp(vals - vals[0:1, :])
    s = e / jnp.sum(e, axis=0, keepdims=True)

    idx_ref[...] = idxs
    scr_ref[...] = s


def _hist_sc_body(idx_hbm, scr_hbm, out_hbm, idx_v, scr_v, himp, hld, outv):
    tpw = idx_v.shape[1]
    wid = lax.axis_index("s") * _SC_CORES + lax.axis_index("c")
    base = wid * tpw
    pltpu.sync_copy(idx_hbm.at[:, pl.ds(base, tpw)], idx_v)
    pltpu.sync_copy(scr_hbm.at[:, pl.ds(base, tpw)], scr_v)
    zeros16 = jnp.zeros((_LANES,), jnp.float32)
    ones16 = jnp.ones((_LANES,), jnp.float32)
    for j in range(_HIST // _LANES):
        himp[pl.ds(j * _LANES, _LANES)] = zeros16
        hld[pl.ds(j * _LANES, _LANES)] = zeros16
    row_base = lax.iota(jnp.int32, _LANES) * N_EXPERTS

    def body(i, carry):
        off = i * _LANES
        for k in range(NUM_SELECTS):
            iv = idx_v[k, pl.ds(off, _LANES)]
            sv = scr_v[k, pl.ds(off, _LANES)]
            addr = row_base + iv
            plsc.addupdate_scatter(himp, [addr], sv)
            plsc.addupdate_scatter(
                hld, [addr], jnp.where(sv > 0.0, ones16, zeros16))
        return carry

    lax.fori_loop(0, tpw // _LANES, body, 0)

    for c in range(N_EXPERTS // _LANES):
        acc_i = jnp.zeros((_LANES,), jnp.float32)
        acc_l = jnp.zeros((_LANES,), jnp.float32)
        for r in range(_LANES):
            acc_i = acc_i + himp[pl.ds(r * N_EXPERTS + c * _LANES, _LANES)]
            acc_l = acc_l + hld[pl.ds(r * N_EXPERTS + c * _LANES, _LANES)]
        outv[pl.ds(c * _LANES, _LANES)] = acc_i
        outv[pl.ds(N_EXPERTS + c * _LANES, _LANES)] = acc_l
    pltpu.sync_copy(outv, out_hbm.at[wid])


def _loss_body(p_ref, loss_ref):
    imp = jnp.sum(p_ref[:, 0:N_EXPERTS], axis=0, keepdims=True)
    ld = jnp.sum(p_ref[:, N_EXPERTS:2 * N_EXPERTS], axis=0, keepdims=True)

    def cv2(v):
        mean = jnp.sum(v, axis=1, keepdims=True) / N_EXPERTS
        var = jnp.sum((v - mean) ** 2, axis=1, keepdims=True) / (N_EXPERTS - 1)
        return var / (mean ** 2 + 1e-10)

    loss_ref[...] = (cv2(imp) + cv2(ld)) * 0.01


@jax.jit
def kernel(x, W1, W2):
    half = N_TOK // 2
    nbh = half // BLOCK_T
    w1t = W1.T

    def gate_half(off):
        return pl.pallas_call(
            _gate_body,
            grid=(nbh,),
            in_specs=[
                pl.BlockSpec((BLOCK_T, D_MODEL), lambda i: (i + off, 0)),
                pl.BlockSpec((D_MODEL, N_EXPERTS), lambda i: (0, 0)),
                pl.BlockSpec((N_EXPERTS, N_EXPERTS), lambda i: (0, 0)),
            ],
            out_specs=[
                pl.BlockSpec((NUM_SELECTS, BLOCK_T), lambda i: (0, i)),
                pl.BlockSpec((NUM_SELECTS, BLOCK_T), lambda i: (0, i)),
            ],
            out_shape=[
                jax.ShapeDtypeStruct((NUM_SELECTS, half), jnp.int32),
                jax.ShapeDtypeStruct((NUM_SELECTS, half), jnp.float32),
            ],
            compiler_params=pltpu.CompilerParams(
                dimension_semantics=("arbitrary",),
            ),
        )(x, w1t, W2)

    def hist_half(idxs_t, scores_t):
        tpw = idxs_t.shape[1] // _SC_WORKERS
        return pl.kernel(
            _hist_sc_body,
            mesh=plsc.VectorSubcoreMesh(core_axis_name="c",
                                        subcore_axis_name="s"),
            out_type=jax.ShapeDtypeStruct((_SC_WORKERS, 2 * N_EXPERTS),
                                          jnp.float32),
            scratch_types=[
                pltpu.VMEM((NUM_SELECTS, tpw), jnp.int32),
                pltpu.VMEM((NUM_SELECTS, tpw), jnp.float32),
                pltpu.VMEM((_HIST,), jnp.float32),
                pltpu.VMEM((_HIST,), jnp.float32),
                pltpu.VMEM((2 * N_EXPERTS,), jnp.float32),
            ],
            compiler_params=pltpu.CompilerParams(needs_layout_passes=False),
        )(idxs_t, scores_t)

    i0, s0 = gate_half(0)
    i1, s1 = gate_half(nbh)
    p0 = hist_half(i0, s0)
    p1 = hist_half(i1, s1)

    loss = pl.pallas_call(
        _loss_body,
        out_shape=jax.ShapeDtypeStruct((1, 1), jnp.float32),
    )(jnp.concatenate([p0, p1], axis=0))

    idxs = jnp.concatenate([i0.T, i1.T], axis=0)
    scores = jnp.concatenate([s0.T, s1.T], axis=0)
    return idxs, scores, jnp.reshape(loss, ())


# speedup vs baseline: 1.0747x; 1.0747x over previous
import functools

import jax
import jax.numpy as jnp
from jax import lax
from jax.experimental import pallas as pl
from jax.experimental.pallas import tpu as pltpu
from jax.experimental.pallas import tpu_sc as plsc

N_TOK = 32768
D_MODEL = 1024
N_EXPERTS = 64
NUM_SELECTS = 8
BLOCK_T = 4096

_SC_CORES = 2
_SC_SUBCORES = 16
_SC_WORKERS = _SC_CORES * _SC_SUBCORES
_TOK_PER_W = N_TOK // _SC_WORKERS
_PAIRS_PER_W = _TOK_PER_W * NUM_SELECTS
_LANES = 16
_HIST = _LANES * N_EXPERTS


def _gate_body(x_ref, w1_ref, w2_ref, idx_ref, scr_ref):
    h = jnp.tanh(jnp.dot(x_ref[...], w1_ref[...],
                         preferred_element_type=jnp.float32))
    lt = jax.lax.dot_general(w2_ref[...], h, (((1,), (1,)), ((), ())),
                             preferred_element_type=jnp.float32)
    ne, bt = lt.shape
    iota_e = jax.lax.broadcasted_iota(jnp.int32, (ne, bt), 0)

    l = lt
    idx_rows, val_rows = [], []
    for _ in range(NUM_SELECTS):
        m = jnp.max(l, axis=0, keepdims=True)
        cand = jnp.where(l == m, iota_e, ne)
        idx = jnp.min(cand, axis=0, keepdims=True)
        onehot = iota_e == idx
        idx_rows.append(idx)
        val_rows.append(m)
        l = jnp.where(onehot, -jnp.inf, l)

    vals = jnp.concatenate(val_rows, axis=0)
    idxs = jnp.concatenate(idx_rows, axis=0)
    e = jnp.exp(vals - vals[0:1, :])
    s = e / jnp.sum(e, axis=0, keepdims=True)

    idx_ref[...] = idxs
    scr_ref[...] = s


def _hist_sc_body(idx_hbm, scr_hbm, out_hbm, idx_v, scr_v, himp, hld, outv):
    tpw = idx_v.shape[1]
    wid = lax.axis_index("s") * _SC_CORES + lax.axis_index("c")
    base = wid * tpw
    pltpu.sync_copy(idx_hbm.at[:, pl.ds(base, tpw)], idx_v)
    pltpu.sync_copy(scr_hbm.at[:, pl.ds(base, tpw)], scr_v)
    zeros16 = jnp.zeros((_LANES,), jnp.float32)
    ones16 = jnp.ones((_LANES,), jnp.float32)
    for j in range(_HIST // _LANES):
        himp[pl.ds(j * _LANES, _LANES)] = zeros16
        hld[pl.ds(j * _LANES, _LANES)] = zeros16
    row_base = lax.iota(jnp.int32, _LANES) * N_EXPERTS

    def body(i, carry):
        off = i * _LANES
        for k in range(NUM_SELECTS):
            iv = idx_v[k, pl.ds(off, _LANES)]
            sv = scr_v[k, pl.ds(off, _LANES)]
            addr = row_base + iv
            plsc.addupdate_scatter(himp, [addr], sv)
            plsc.addupdate_scatter(
                hld, [addr], jnp.where(sv > 0.0, ones16, zeros16))
        return carry

    lax.fori_loop(0, tpw // _LANES, body, 0)

    for c in range(N_EXPERTS // _LANES):
        acc_i = jnp.zeros((_LANES,), jnp.float32)
        acc_l = jnp.zeros((_LANES,), jnp.float32)
        for r in range(_LANES):
            acc_i = acc_i + himp[pl.ds(r * N_EXPERTS + c * _LANES, _LANES)]
            acc_l = acc_l + hld[pl.ds(r * N_EXPERTS + c * _LANES, _LANES)]
        outv[pl.ds(c * _LANES, _LANES)] = acc_i
        outv[pl.ds(N_EXPERTS + c * _LANES, _LANES)] = acc_l
    pltpu.sync_copy(outv, out_hbm.at[wid])


def _loss_body(p_ref, loss_ref):
    imp = jnp.sum(p_ref[:, 0:N_EXPERTS], axis=0, keepdims=True)
    ld = jnp.sum(p_ref[:, N_EXPERTS:2 * N_EXPERTS], axis=0, keepdims=True)

    def cv2(v):
        mean = jnp.sum(v, axis=1, keepdims=True) / N_EXPERTS
        var = jnp.sum((v - mean) ** 2, axis=1, keepdims=True) / (N_EXPERTS - 1)
        return var / (mean ** 2 + 1e-10)

    loss_ref[...] = (cv2(imp) + cv2(ld)) * 0.01


@jax.jit
def kernel(x, W1, W2):
    nb = N_TOK // BLOCK_T
    idxs_t, scores_t = pl.pallas_call(
        _gate_body,
        grid=(nb,),
        in_specs=[
            pl.BlockSpec((BLOCK_T, D_MODEL), lambda i: (i, 0)),
            pl.BlockSpec((D_MODEL, N_EXPERTS), lambda i: (0, 0)),
            pl.BlockSpec((N_EXPERTS, N_EXPERTS), lambda i: (0, 0)),
        ],
        out_specs=[
            pl.BlockSpec((NUM_SELECTS, BLOCK_T), lambda i: (0, i)),
            pl.BlockSpec((NUM_SELECTS, BLOCK_T), lambda i: (0, i)),
        ],
        out_shape=[
            jax.ShapeDtypeStruct((NUM_SELECTS, N_TOK), jnp.int32),
            jax.ShapeDtypeStruct((NUM_SELECTS, N_TOK), jnp.float32),
        ],
        compiler_params=pltpu.CompilerParams(
            dimension_semantics=("arbitrary",),
        ),
    )(x, W1.T, W2)

    partials = pl.kernel(
        _hist_sc_body,
        mesh=plsc.VectorSubcoreMesh(core_axis_name="c", subcore_axis_name="s"),
        out_type=jax.ShapeDtypeStruct((_SC_WORKERS, 2 * N_EXPERTS),
                                      jnp.float32),
        scratch_types=[
            pltpu.VMEM((NUM_SELECTS, _TOK_PER_W), jnp.int32),
            pltpu.VMEM((NUM_SELECTS, _TOK_PER_W), jnp.float32),
            pltpu.VMEM((_HIST,), jnp.float32),
            pltpu.VMEM((_HIST,), jnp.float32),
            pltpu.VMEM((2 * N_EXPERTS,), jnp.float32),
        ],
        compiler_params=pltpu.CompilerParams(needs_layout_passes=False),
    )(idxs_t, scores_t)

    loss = pl.pallas_call(
        _loss_body,
        out_shape=jax.ShapeDtypeStruct((1, 1), jnp.float32),
    )(partials)

    return idxs_t.T, scores_t.T, jnp.reshape(loss, ())


# speedup vs baseline: 1.0999x; 1.0234x over previous
import functools

import jax
import jax.numpy as jnp
from jax import lax
from jax.experimental import pallas as pl
from jax.experimental.pallas import tpu as pltpu
from jax.experimental.pallas import tpu_sc as plsc

N_TOK = 32768
D_MODEL = 1024
N_EXPERTS = 64
NUM_SELECTS = 8
BLOCK_T = 4096

_SC_CORES = 2
_SC_SUBCORES = 16
_SC_WORKERS = _SC_CORES * _SC_SUBCORES
_TOK_PER_W = N_TOK // _SC_WORKERS
_PAIRS_PER_W = _TOK_PER_W * NUM_SELECTS
_LANES = 16
_HIST = _LANES * N_EXPERTS


def _gate_body(x_ref, w1_ref, w2_ref, idx_ref, scr_ref):
    h = jnp.tanh(jnp.dot(x_ref[...], w1_ref[...],
                         preferred_element_type=jnp.float32))
    lt = jax.lax.dot_general(w2_ref[...], h, (((1,), (1,)), ((), ())),
                             preferred_element_type=jnp.float32)
    ne, bt = lt.shape
    iota_e = jax.lax.broadcasted_iota(jnp.int32, (ne, bt), 0)

    l = lt
    idx_rows, val_rows = [], []
    for _ in range(NUM_SELECTS):
        m = jnp.max(l, axis=0, keepdims=True)
        cand = jnp.where(l == m, iota_e, ne)
        idx = jnp.min(cand, axis=0, keepdims=True)
        onehot = iota_e == idx
        idx_rows.append(idx)
        val_rows.append(m)
        l = jnp.where(onehot, -jnp.inf, l)

    vals = jnp.concatenate(val_rows, axis=0)
    idxs = jnp.concatenate(idx_rows, axis=0)
    e = jnp.exp(vals - vals[0:1, :])
    s = e / jnp.sum(e, axis=0, keepdims=True)

    idx_ref[...] = idxs
    scr_ref[...] = s


def _hist_sc_body(idx_hbm, scr_hbm, out_hbm, idx_v, scr_v, himp, hld, outv):
    tpw = idx_v.shape[1]
    wid = lax.axis_index("s") * _SC_CORES + lax.axis_index("c")
    base = wid * tpw
    pltpu.sync_copy(idx_hbm.at[:, pl.ds(base, tpw)], idx_v)
    pltpu.sync_copy(scr_hbm.at[:, pl.ds(base, tpw)], scr_v)
    zeros16 = jnp.zeros((_LANES,), jnp.float32)
    ones16 = jnp.ones((_LANES,), jnp.float32)
    for j in range(_HIST // _LANES):
        himp[pl.ds(j * _LANES, _LANES)] = zeros16
        hld[pl.ds(j * _LANES, _LANES)] = zeros16
    row_base = lax.iota(jnp.int32, _LANES) * N_EXPERTS

    @plsc.parallel_loop(0, tpw // _LANES, unroll=4)
    def _scatter(i):
        off = i * _LANES
        for k in range(NUM_SELECTS):
            iv = idx_v[k, pl.ds(off, _LANES)]
            sv = scr_v[k, pl.ds(off, _LANES)]
            addr = row_base + iv
            plsc.addupdate_scatter(himp, [addr], sv)
            plsc.addupdate_scatter(
                hld, [addr], jnp.where(sv > 0.0, ones16, zeros16))

    for c in range(N_EXPERTS // _LANES):
        acc_i = jnp.zeros((_LANES,), jnp.float32)
        acc_l = jnp.zeros((_LANES,), jnp.float32)
        for r in range(_LANES):
            acc_i = acc_i + himp[pl.ds(r * N_EXPERTS + c * _LANES, _LANES)]
            acc_l = acc_l + hld[pl.ds(r * N_EXPERTS + c * _LANES, _LANES)]
        outv[pl.ds(c * _LANES, _LANES)] = acc_i
        outv[pl.ds(N_EXPERTS + c * _LANES, _LANES)] = acc_l
    pltpu.sync_copy(outv, out_hbm.at[wid])


def _loss_body(p_ref, loss_ref):
    imp = jnp.sum(p_ref[:, 0:N_EXPERTS], axis=0, keepdims=True)
    ld = jnp.sum(p_ref[:, N_EXPERTS:2 * N_EXPERTS], axis=0, keepdims=True)

    def cv2(v):
        mean = jnp.sum(v, axis=1, keepdims=True) / N_EXPERTS
        var = jnp.sum((v - mean) ** 2, axis=1, keepdims=True) / (N_EXPERTS - 1)
        return var / (mean ** 2 + 1e-10)

    loss_ref[...] = (cv2(imp) + cv2(ld)) * 0.01


@jax.jit
def kernel(x, W1, W2):
    nb = N_TOK // BLOCK_T
    idxs_t, scores_t = pl.pallas_call(
        _gate_body,
        grid=(nb,),
        in_specs=[
            pl.BlockSpec((BLOCK_T, D_MODEL), lambda i: (i, 0)),
            pl.BlockSpec((D_MODEL, N_EXPERTS), lambda i: (0, 0)),
            pl.BlockSpec((N_EXPERTS, N_EXPERTS), lambda i: (0, 0)),
        ],
        out_specs=[
            pl.BlockSpec((NUM_SELECTS, BLOCK_T), lambda i: (0, i)),
            pl.BlockSpec((NUM_SELECTS, BLOCK_T), lambda i: (0, i)),
        ],
        out_shape=[
            jax.ShapeDtypeStruct((NUM_SELECTS, N_TOK), jnp.int32),
            jax.ShapeDtypeStruct((NUM_SELECTS, N_TOK), jnp.float32),
        ],
        compiler_params=pltpu.CompilerParams(
            dimension_semantics=("arbitrary",),
        ),
    )(x, W1.T, W2)

    partials = pl.kernel(
        _hist_sc_body,
        mesh=plsc.VectorSubcoreMesh(core_axis_name="c", subcore_axis_name="s"),
        out_type=jax.ShapeDtypeStruct((_SC_WORKERS, 2 * N_EXPERTS),
                                      jnp.float32),
        scratch_types=[
            pltpu.VMEM((NUM_SELECTS, _TOK_PER_W), jnp.int32),
            pltpu.VMEM((NUM_SELECTS, _TOK_PER_W), jnp.float32),
            pltpu.VMEM((_HIST,), jnp.float32),
            pltpu.VMEM((_HIST,), jnp.float32),
            pltpu.VMEM((2 * N_EXPERTS,), jnp.float32),
        ],
        compiler_params=pltpu.CompilerParams(needs_layout_passes=False),
    )(idxs_t, scores_t)

    loss = pl.pallas_call(
        _loss_body,
        out_shape=jax.ShapeDtypeStruct((1, 1), jnp.float32),
    )(partials)

    return idxs_t.T, scores_t.T, jnp.reshape(loss, ())
